# baseline (speedup 1.0000x reference)
import functools

import jax
import jax.numpy as jnp
from jax import lax
from jax.experimental import pallas as pl
from jax.experimental.pallas import tpu as pltpu
from jax.experimental.pallas import tpu_sc as plsc

B = 32
WIN = 8
NODE_COUNT = 40
D_IN = 256
H = 8
DH = 64
D_OUT = H * DH
HID = 128
E = 160000
NP = 2560
N = B * WIN * NODE_COUNT
NROW = 2 * NP
NG = B * WIN


def _proj_body(x_ref, w_ref, alf_ref, arf_ref, feat_ref, el_ref, er_ref):
    x = x_ref[...]
    w = w_ref[0]
    feat = jnp.dot(x, w, preferred_element_type=jnp.float32)
    feat_ref[...] = feat
    seg = (lax.broadcasted_iota(jnp.int32, (D_OUT, H), 0) // DH
           == lax.broadcasted_iota(jnp.int32, (D_OUT, H), 1)).astype(jnp.float32)
    el_ref[...] = jnp.dot(feat * alf_ref[0, 0][None, :], seg,
                          preferred_element_type=jnp.float32)
    er_ref[...] = jnp.dot(feat * arf_ref[0, 0][None, :], seg,
                          preferred_element_type=jnp.float32)


def _project(packed, wstk, alf, arf):
    nblk = NROW // 256
    return pl.pallas_call(
        _proj_body,
        grid=(nblk,),
        in_specs=[
            pl.BlockSpec((256, D_IN), lambda i: (i, 0)),
            pl.BlockSpec((1, D_IN, D_OUT), lambda i: (i // 10, 0, 0)),
            pl.BlockSpec((1, 1, D_OUT), lambda i: (i // 10, 0, 0)),
            pl.BlockSpec((1, 1, D_OUT), lambda i: (i // 10, 0, 0)),
        ],
        out_specs=[
            pl.BlockSpec((256, D_OUT), lambda i: (i, 0)),
            pl.BlockSpec((256, H), lambda i: (i, 0)),
            pl.BlockSpec((256, H), lambda i: (i, 0)),
        ],
        out_shape=[
            jax.ShapeDtypeStruct((NROW, D_OUT), jnp.float32),
            jax.ShapeDtypeStruct((NROW, H), jnp.float32),
            jax.ShapeDtypeStruct((NROW, H), jnp.float32),
        ],
    )(packed, wstk, alf, arf)


def _gat_body(cnt_ref, er_ref, elt_ref, feat_ref, out_ref):
    cnt = cnt_ref[...]
    for h in range(H):
        e = er_ref[:, h][:, None] + elt_ref[0, h, :][None, :]
        e = jnp.maximum(e, 0.2 * e)
        p = cnt * jnp.exp(e)
        esum = jnp.sum(p, axis=1) + 1e-9
        acc = jnp.dot(p, feat_ref[:, h * DH:(h + 1) * DH],
                      preferred_element_type=jnp.float32)
        o = acc / esum[:, None]
        out_ref[:, h * DH:(h + 1) * DH] = jnp.where(o > 0, o, jnp.exp(o) - 1.0)


def _gat_dense(cnt2, er, elt, feat):
    return pl.pallas_call(
        _gat_body,
        grid=(2, NP // 256),
        in_specs=[
            pl.BlockSpec((256, NP), lambda g, d: (g * 10 + d, 0)),
            pl.BlockSpec((256, H), lambda g, d: (g * 10 + d, 0)),
            pl.BlockSpec((1, H, NP), lambda g, d: (g, 0, 0)),
            pl.BlockSpec((NP, D_OUT), lambda g, d: (g, 0)),
        ],
        out_specs=pl.BlockSpec((256, D_OUT), lambda g, d: (g * 10 + d, 0)),
        out_shape=jax.ShapeDtypeStruct((NROW, D_OUT), jnp.float32),
    )(cnt2, er, elt, feat)


def _final_body(gat_ref, bnd_ref, w1_ref, b1_ref, w2_ref, out_ref):
    iota_p = lax.broadcasted_iota(jnp.int32, (NG, NP), 1)
    au = bnd_ref[0, :][:, None]
    bu = bnd_ref[1, :][:, None]
    ai = bnd_ref[2, :][:, None]
    bi = bnd_ref[3, :][:, None]
    mu = ((iota_p >= au) & (iota_p < bu)).astype(jnp.float32)
    mi = ((iota_p >= ai) & (iota_p < bi)).astype(jnp.float32)
    snaps = (jnp.dot(mu, gat_ref[:NP, :], preferred_element_type=jnp.float32)
             + jnp.dot(mi, gat_ref[NP:, :], preferred_element_type=jnp.float32)
             ) * (1.0 / NODE_COUNT)
    q = jnp.tanh(jnp.dot(snaps, w1_ref[...], preferred_element_type=jnp.float32)
                 + b1_ref[...])
    s = jnp.sum(q * w2_ref[...], axis=1)
    er_ = jnp.exp(s)[None, :]
    iota_r = lax.broadcasted_iota(jnp.int32, (B, NG), 1)
    iota_b = lax.broadcasted_iota(jnp.int32, (B, NG), 0)
    bsel = jnp.where(iota_r // WIN == iota_b, er_, 0.0)
    denom = jnp.sum(bsel, axis=1)
    out_ref[...] = jnp.dot(bsel, snaps,
                           preferred_element_type=jnp.float32) / denom[:, None]


def _final(gat, bounds, w1, b1row, w2row):
    return pl.pallas_call(
        _final_body,
        grid=(1,),
        in_specs=[
            pl.BlockSpec((NROW, D_OUT), lambda i: (0, 0)),
            pl.BlockSpec((4, NG), lambda i: (0, 0)),
            pl.BlockSpec((D_OUT, HID), lambda i: (0, 0)),
            pl.BlockSpec((1, HID), lambda i: (0, 0)),
            pl.BlockSpec((1, HID), lambda i: (0, 0)),
        ],
        out_specs=pl.BlockSpec((B, D_OUT), lambda i: (0, 0)),
        out_shape=jax.ShapeDtypeStruct((B, D_OUT), jnp.float32),
    )(gat, bounds, w1, b1row, w2row)


_NSL = 8
_QROWS = NP // _NSL
_QW = _QROWS * NP
_SH = _QW + 128
_DUMP = _QW
_EPT = E // 16
_NB = 79
_TSL = _QW // 16
_ZCH = 6400


def _sc_body(table, gidx, src_u, dst_u, src_i, dst_i,
             packed, cnt,
             idx_a, rows_v, dstb, srcb, idx_buf, ones_v, zbuf,
             shared, sem_g, sem_s, sem_z):
    cid = lax.axis_index("c")
    sid = lax.axis_index("s")
    wid = sid * 2 + cid

    def _fill_ones(j, _):
        ones_v[pl.ds(j * 16, 16)] = jnp.full((16,), 1.0, jnp.float32)
        return _
    lax.fori_loop(0, _NB * 8, _fill_ones, 0)

    def _fill_z(j, _):
        zbuf[pl.ds(j * 16, 16)] = jnp.zeros((16,), jnp.float32)
        return _
    lax.fori_loop(0, _ZCH // 16, _fill_z, 0)

    lanes = lax.broadcasted_iota(jnp.int32, (16,), 0)

    for t in range((_NB * 128 - _EPT) // 16):
        dstb[pl.ds(_EPT + t * 16, 16)] = jnp.full((16,), -1, jnp.int32)

    for g in range(2):
        dsrc = dst_u if g == 0 else dst_i
        ssrc = src_u if g == 0 else src_i
        pltpu.sync_copy(dsrc.at[pl.ds(sid * _EPT, _EPT)],
                        dstb.at[pl.ds(0, _EPT)])
        pltpu.sync_copy(ssrc.at[pl.ds(sid * _EPT, _EPT)],
                        srcb.at[pl.ds(0, _EPT)])
        for oct_ in range(_NSL // 2):
            q = (_NSL // 2) * cid + oct_
            qlo = q * _QROWS
            nz = _TSL // _ZCH
            for z in range(nz):
                pltpu.async_copy(
                    zbuf, shared.at[pl.ds(sid * _TSL + z * _ZCH, _ZCH)],
                    sem_z)
            for z in range(nz):
                pltpu.make_async_copy(
                    zbuf, shared.at[pl.ds(sid * _TSL, _ZCH)], sem_z).wait()

            def _mk(j, _, qlo=qlo):
                for k in range(8):
                    p0 = j * 128 + k * 16
                    d = dstb[pl.ds(p0, 16)]
                    s = srcb[pl.ds(p0, 16)]
                    ok = (d >= qlo) & (d < qlo + _QROWS)
                    flat = (d - qlo) * NP + s
                    dumpv = (_DUMP + k * 16) + lanes
                    idx_buf[pl.ds(p0, 16)] = jnp.where(ok, flat, dumpv)
                return _
            lax.fori_loop(0, _NB, _mk, 0)
            plsc.subcore_barrier()
            pltpu.async_copy(ones_v, shared.at[idx_buf], sem_s,
                             add=True).wait()
            plsc.subcore_barrier()
            doff = (g * NP + qlo) * NP + sid * _TSL
            pltpu.sync_copy(shared.at[pl.ds(sid * _TSL, _TSL)],
                            cnt.at[pl.ds(doff, _TSL)])
            plsc.subcore_barrier()

    for ph in range(2):
        gbase = wid * 160 + ph * 80
        pltpu.sync_copy(gidx.at[pl.ds(gbase, 80)], idx_a)
        pltpu.async_copy(table.at[idx_a], rows_v, sem_g).wait()
        pltpu.sync_copy(rows_v, packed.at[pl.ds(gbase, 80)])


def _pack_and_count(table, gidx, ei_u, ei_i):
    mesh = plsc.VectorSubcoreMesh(core_axis_name="c", subcore_axis_name="s")
    f = pl.kernel(
        _sc_body,
        out_type=[
            jax.ShapeDtypeStruct((NROW, D_IN), jnp.float32),
            jax.ShapeDtypeStruct((2 * NP * NP,), jnp.float32),
        ],
        mesh=mesh,
        scratch_types=[
            pltpu.VMEM((80,), jnp.int32),
            pltpu.VMEM((80, D_IN), jnp.float32),
            pltpu.VMEM((_NB * 128,), jnp.int32),
            pltpu.VMEM((_NB * 128,), jnp.int32),
            pltpu.VMEM((_NB * 128,), jnp.int32),
            pltpu.VMEM((_NB * 128,), jnp.float32),
            pltpu.VMEM((_ZCH,), jnp.float32),
            pltpu.VMEM_SHARED((_SH,), jnp.float32),
            pltpu.SemaphoreType.DMA,
            pltpu.SemaphoreType.DMA,
            pltpu.SemaphoreType.DMA,
        ],
    )
    packed, cnt = f(table, gidx, ei_u[0], ei_u[1], ei_i[0], ei_i[1])
    return packed, cnt.reshape(NROW, NP)


def kernel(outputs, type_edges, edge_index_user, edge_index_item,
           hop_embedding, Wg_u, al_u, ar_u, Wg_i, al_i, ar_i,
           sa_u_w1, sa_u_b1, sa_u_w2, sa_i_w1, sa_i_b1, sa_i_w2,
           sn_w1, sn_b1, sn_w2):
    types = type_edges.reshape(-1).astype(jnp.int32)
    mask_u = types == 0
    mask_i = types == 1
    cu = jnp.cumsum(mask_u.astype(jnp.int32))
    ci = jnp.cumsum(mask_i.astype(jnp.int32))
    q = jnp.arange(1, NP + 1, dtype=jnp.int32)
    gidx_u = jnp.searchsorted(cu, q, side="left").astype(jnp.int32)
    gidx_i = jnp.searchsorted(ci, q, side="left").astype(jnp.int32)
    gidx = jnp.concatenate([gidx_u, gidx_i])

    table = jnp.concatenate(
        [outputs.reshape(-1, D_IN), jnp.zeros((8, D_IN), jnp.float32)])

    cu0 = jnp.concatenate([jnp.zeros((1,), jnp.int32), cu])
    ci0 = jnp.concatenate([jnp.zeros((1,), jnp.int32), ci])
    bounds = jnp.stack([
        cu0[:-1:NODE_COUNT], cu0[NODE_COUNT::NODE_COUNT],
        ci0[:-1:NODE_COUNT], ci0[NODE_COUNT::NODE_COUNT]])

    packed, cnt2 = _pack_and_count(
        table, gidx, edge_index_user.astype(jnp.int32),
        edge_index_item.astype(jnp.int32))

    wstk = jnp.stack([Wg_u, Wg_i])
    alf = jnp.stack([al_u.reshape(1, -1), al_i.reshape(1, -1)])
    arf = jnp.stack([ar_u.reshape(1, -1), ar_i.reshape(1, -1)])

    feat, el, er = _project(packed, wstk, alf, arf)
    elt = el.reshape(2, NP, H).transpose(0, 2, 1)
    gat = _gat_dense(cnt2, er, elt, feat)
    return _final(gat, bounds, sn_w1, sn_b1.reshape(1, HID),
                  sn_w2.reshape(1, HID))


# speedup vs baseline: 1.7206x; 1.7206x over previous
import functools

import jax
import jax.numpy as jnp
from jax import lax
from jax.experimental import pallas as pl
from jax.experimental.pallas import tpu as pltpu
from jax.experimental.pallas import tpu_sc as plsc

B = 32
WIN = 8
NODE_COUNT = 40
D_IN = 256
H = 8
DH = 64
D_OUT = H * DH
HID = 128
E = 160000
NP = 2560
N = B * WIN * NODE_COUNT
NROW = 2 * NP
NG = B * WIN


def _proj_body(x_ref, w_ref, alf_ref, arf_ref, feat_ref, el_ref, er_ref):
    x = x_ref[...]
    w = w_ref[0]
    feat = jnp.dot(x, w, preferred_element_type=jnp.float32)
    feat_ref[...] = feat
    seg = (lax.broadcasted_iota(jnp.int32, (D_OUT, H), 0) // DH
           == lax.broadcasted_iota(jnp.int32, (D_OUT, H), 1)).astype(jnp.float32)
    el_ref[...] = jnp.dot(feat * alf_ref[0, 0][None, :], seg,
                          preferred_element_type=jnp.float32)
    er_ref[...] = jnp.dot(feat * arf_ref[0, 0][None, :], seg,
                          preferred_element_type=jnp.float32)


def _project(packed, wstk, alf, arf):
    nblk = NROW // 256
    return pl.pallas_call(
        _proj_body,
        grid=(nblk,),
        in_specs=[
            pl.BlockSpec((256, D_IN), lambda i: (i, 0)),
            pl.BlockSpec((1, D_IN, D_OUT), lambda i: (i // 10, 0, 0)),
            pl.BlockSpec((1, 1, D_OUT), lambda i: (i // 10, 0, 0)),
            pl.BlockSpec((1, 1, D_OUT), lambda i: (i // 10, 0, 0)),
        ],
        out_specs=[
            pl.BlockSpec((256, D_OUT), lambda i: (i, 0)),
            pl.BlockSpec((256, H), lambda i: (i, 0)),
            pl.BlockSpec((256, H), lambda i: (i, 0)),
        ],
        out_shape=[
            jax.ShapeDtypeStruct((NROW, D_OUT), jnp.float32),
            jax.ShapeDtypeStruct((NROW, H), jnp.float32),
            jax.ShapeDtypeStruct((NROW, H), jnp.float32),
        ],
    )(packed, wstk, alf, arf)


def _gat_body(cnt_ref, er_ref, elt_ref, feat_ref, out_ref):
    cnt = cnt_ref[...]
    for h in range(H):
        e = er_ref[:, h][:, None] + elt_ref[0, h, :][None, :]
        e = jnp.maximum(e, 0.2 * e)
        p = cnt * jnp.exp(e)
        esum = jnp.sum(p, axis=1) + 1e-9
        acc = jnp.dot(p, feat_ref[:, h * DH:(h + 1) * DH],
                      preferred_element_type=jnp.float32)
        o = acc / esum[:, None]
        out_ref[:, h * DH:(h + 1) * DH] = jnp.where(o > 0, o, jnp.exp(o) - 1.0)


def _gat_dense(cnt2, er, elt, feat):
    return pl.pallas_call(
        _gat_body,
        grid=(2, NP // 256),
        in_specs=[
            pl.BlockSpec((256, NP), lambda g, d: (g * 10 + d, 0)),
            pl.BlockSpec((256, H), lambda g, d: (g * 10 + d, 0)),
            pl.BlockSpec((1, H, NP), lambda g, d: (g, 0, 0)),
            pl.BlockSpec((NP, D_OUT), lambda g, d: (g, 0)),
        ],
        out_specs=pl.BlockSpec((256, D_OUT), lambda g, d: (g * 10 + d, 0)),
        out_shape=jax.ShapeDtypeStruct((NROW, D_OUT), jnp.float32),
    )(cnt2, er, elt, feat)


def _final_body(gat_ref, bnd_ref, w1_ref, b1_ref, w2_ref, out_ref):
    iota_p = lax.broadcasted_iota(jnp.int32, (NG, NP), 1)
    au = bnd_ref[0, :][:, None]
    bu = bnd_ref[1, :][:, None]
    ai = bnd_ref[2, :][:, None]
    bi = bnd_ref[3, :][:, None]
    mu = ((iota_p >= au) & (iota_p < bu)).astype(jnp.float32)
    mi = ((iota_p >= ai) & (iota_p < bi)).astype(jnp.float32)
    snaps = (jnp.dot(mu, gat_ref[:NP, :], preferred_element_type=jnp.float32)
             + jnp.dot(mi, gat_ref[NP:, :], preferred_element_type=jnp.float32)
             ) * (1.0 / NODE_COUNT)
    q = jnp.tanh(jnp.dot(snaps, w1_ref[...], preferred_element_type=jnp.float32)
                 + b1_ref[...])
    s = jnp.sum(q * w2_ref[...], axis=1)
    er_ = jnp.exp(s)[None, :]
    iota_r = lax.broadcasted_iota(jnp.int32, (B, NG), 1)
    iota_b = lax.broadcasted_iota(jnp.int32, (B, NG), 0)
    bsel = jnp.where(iota_r // WIN == iota_b, er_, 0.0)
    denom = jnp.sum(bsel, axis=1)
    out_ref[...] = jnp.dot(bsel, snaps,
                           preferred_element_type=jnp.float32) / denom[:, None]


def _final(gat, bounds, w1, b1row, w2row):
    return pl.pallas_call(
        _final_body,
        grid=(1,),
        in_specs=[
            pl.BlockSpec((NROW, D_OUT), lambda i: (0, 0)),
            pl.BlockSpec((4, NG), lambda i: (0, 0)),
            pl.BlockSpec((D_OUT, HID), lambda i: (0, 0)),
            pl.BlockSpec((1, HID), lambda i: (0, 0)),
            pl.BlockSpec((1, HID), lambda i: (0, 0)),
        ],
        out_specs=pl.BlockSpec((B, D_OUT), lambda i: (0, 0)),
        out_shape=jax.ShapeDtypeStruct((B, D_OUT), jnp.float32),
    )(gat, bounds, w1, b1row, w2row)


_NSL = 8
_QROWS = NP // _NSL
_QW = _QROWS * NP
_SH = _QW + 128
_DUMP = _QW
_EPT = E // 16
_NB = 79
_TSL = _QW // 16
_ZCH = 6400


def _sc_body(table, types, cu_in, src_u, dst_u, src_i, dst_i,
             packed, cnt,
             idx_a, rows_v, dstb, srcb, idx_buf, ones_v, zbuf,
             tvb, cuv, sentb, sidx, svals,
             shared, shared2, sem_g, sem_s, sem_z):
    cid = lax.axis_index("c")
    sid = lax.axis_index("s")

    lanes = lax.broadcasted_iota(jnp.int32, (16,), 0)

    pltpu.sync_copy(types.at[pl.ds(sid * 640, 640)], tvb)
    pltpu.sync_copy(cu_in.at[pl.ds(sid * 640, 640)], cuv)

    for v in range(11):
        sentb[pl.ds(v * 16, 16)] = jnp.full((16,), N, jnp.int32)
    pltpu.sync_copy(sentb, shared2.at[pl.ds(256 + sid * 176, 176)])
    plsc.subcore_barrier()
    is_u = cid == 0
    tsel = jnp.where(is_u, 0, 1)
    for v in range(40):
        t = tvb[pl.ds(v * 16, 16)]
        cug = cuv[pl.ds(v * 16, 16)]
        nvec = (sid * 640 + v * 16) + lanes
        pos = jnp.where(is_u, cug - 1, nvec - cug)
        ok = (t == tsel) & (pos < NP)
        dumpv = (256 + NP + (v % 8) * 16) + lanes
        sidx[v // 8, pl.ds((v % 8) * 16, 16)] = jnp.where(ok, 256 + pos,
                                                          dumpv)
        svals[v // 8, pl.ds((v % 8) * 16, 16)] = nvec
    for j in range(5):
        pltpu.async_copy(svals.at[j], shared2.at[sidx.at[j]], sem_g)
    for j in range(5):
        pltpu.make_async_copy(svals.at[0], shared2.at[sidx.at[0]],
                              sem_g).wait()
    plsc.subcore_barrier()

    for ph in range(2):
        pltpu.sync_copy(shared2.at[pl.ds(256 + sid * 160 + ph * 80, 80)],
                        idx_a)
        pltpu.async_copy(table.at[idx_a], rows_v, sem_g).wait()
        pltpu.sync_copy(
            rows_v, packed.at[pl.ds(cid * NP + sid * 160 + ph * 80, 80)])

    def _fill_ones(j, _):
        ones_v[pl.ds(j * 16, 16)] = jnp.full((16,), 1.0, jnp.float32)
        return _
    lax.fori_loop(0, _NB * 8, _fill_ones, 0)

    def _fill_z(j, _):
        zbuf[pl.ds(j * 16, 16)] = jnp.zeros((16,), jnp.float32)
        return _
    lax.fori_loop(0, _ZCH // 16, _fill_z, 0)

    for t in range((_NB * 128 - _EPT) // 16):
        dstb[pl.ds(_EPT + t * 16, 16)] = jnp.full((16,), -1, jnp.int32)

    for g in range(2):
        dsrc = dst_u if g == 0 else dst_i
        ssrc = src_u if g == 0 else src_i
        pltpu.sync_copy(dsrc.at[pl.ds(sid * _EPT, _EPT)],
                        dstb.at[pl.ds(0, _EPT)])
        pltpu.sync_copy(ssrc.at[pl.ds(sid * _EPT, _EPT)],
                        srcb.at[pl.ds(0, _EPT)])
        for oct_ in range(_NSL // 2):
            q = (_NSL // 2) * cid + oct_
            qlo = q * _QROWS
            nz = _TSL // _ZCH
            for z in range(nz):
                pltpu.async_copy(
                    zbuf, shared.at[pl.ds(sid * _TSL + z * _ZCH, _ZCH)],
                    sem_z)
            for z in range(nz):
                pltpu.make_async_copy(
                    zbuf, shared.at[pl.ds(sid * _TSL, _ZCH)], sem_z).wait()

            def _mk(j, _, qlo=qlo):
                for k in range(8):
                    p0 = j * 128 + k * 16
                    d = dstb[pl.ds(p0, 16)]
                    s = srcb[pl.ds(p0, 16)]
                    ok = (d >= qlo) & (d < qlo + _QROWS)
                    flat = (d - qlo) * NP + s
                    dumpv = (_DUMP + k * 16) + lanes
                    idx_buf[pl.ds(p0, 16)] = jnp.where(ok, flat, dumpv)
                return _
            lax.fori_loop(0, _NB, _mk, 0)
            plsc.subcore_barrier()
            pltpu.async_copy(ones_v, shared.at[idx_buf], sem_s,
                             add=True).wait()
            plsc.subcore_barrier()
            doff = (g * NP + qlo) * NP + sid * _TSL
            pltpu.sync_copy(shared.at[pl.ds(sid * _TSL, _TSL)],
                            cnt.at[pl.ds(doff, _TSL)])
            plsc.subcore_barrier()


def _pack_and_count(table, types, cu, ei_u, ei_i):
    mesh = plsc.VectorSubcoreMesh(core_axis_name="c", subcore_axis_name="s")
    f = pl.kernel(
        _sc_body,
        out_type=[
            jax.ShapeDtypeStruct((NROW, D_IN), jnp.float32),
            jax.ShapeDtypeStruct((2 * NP * NP,), jnp.float32),
        ],
        mesh=mesh,
        scratch_types=[
            pltpu.VMEM((80,), jnp.int32),
            pltpu.VMEM((80, D_IN), jnp.float32),
            pltpu.VMEM((_NB * 128,), jnp.int32),
            pltpu.VMEM((_NB * 128,), jnp.int32),
            pltpu.VMEM((_NB * 128,), jnp.int32),
            pltpu.VMEM((_NB * 128,), jnp.float32),
            pltpu.VMEM((_ZCH,), jnp.float32),
            pltpu.VMEM((640,), jnp.int32),
            pltpu.VMEM((640,), jnp.int32),
            pltpu.VMEM((176,), jnp.int32),
            pltpu.VMEM((5, 128), jnp.int32),
            pltpu.VMEM((5, 128), jnp.int32),
            pltpu.VMEM_SHARED((_SH,), jnp.float32),
            pltpu.VMEM_SHARED((3072,), jnp.int32),
            pltpu.SemaphoreType.DMA,
            pltpu.SemaphoreType.DMA,
            pltpu.SemaphoreType.DMA,
        ],
    )
    packed, cnt = f(table, types, cu, ei_u[0], ei_u[1], ei_i[0], ei_i[1])
    return packed, cnt.reshape(NROW, NP)


def kernel(outputs, type_edges, edge_index_user, edge_index_item,
           hop_embedding, Wg_u, al_u, ar_u, Wg_i, al_i, ar_i,
           sa_u_w1, sa_u_b1, sa_u_w2, sa_i_w1, sa_i_b1, sa_i_w2,
           sn_w1, sn_b1, sn_w2):
    types = type_edges.reshape(-1).astype(jnp.int32)
    table = jnp.concatenate(
        [outputs.reshape(-1, D_IN), jnp.zeros((8, D_IN), jnp.float32)])
    cu = jnp.cumsum((types == 0).astype(jnp.int32))

    packed, cnt2 = _pack_and_count(
        table, types, cu, edge_index_user.astype(jnp.int32),
        edge_index_item.astype(jnp.int32))

    cu0 = jnp.concatenate([jnp.zeros((1,), jnp.int32), cu])
    a_u = cu0[:-1:NODE_COUNT]
    b_u = cu0[NODE_COUNT::NODE_COUNT]
    g40 = jnp.arange(NG, dtype=jnp.int32) * NODE_COUNT
    bounds = jnp.stack([a_u, b_u, g40 - a_u, g40 + NODE_COUNT - b_u])

    wstk = jnp.stack([Wg_u, Wg_i])
    alf = jnp.stack([al_u.reshape(1, -1), al_i.reshape(1, -1)])
    arf = jnp.stack([ar_u.reshape(1, -1), ar_i.reshape(1, -1)])

    feat, el, er = _project(packed, wstk, alf, arf)
    elt = el.reshape(2, NP, H).transpose(0, 2, 1)
    gat = _gat_dense(cnt2, er, elt, feat)
    return _final(gat, bounds, sn_w1, sn_b1.reshape(1, HID),
                  sn_w2.reshape(1, HID))


# speedup vs baseline: 1.7316x; 1.0064x over previous
import functools

import jax
import jax.numpy as jnp
from jax import lax
from jax.experimental import pallas as pl
from jax.experimental.pallas import tpu as pltpu
from jax.experimental.pallas import tpu_sc as plsc

B = 32
WIN = 8
NODE_COUNT = 40
D_IN = 256
H = 8
DH = 64
D_OUT = H * DH
HID = 128
E = 160000
NP = 2560
N = B * WIN * NODE_COUNT
NROW = 2 * NP
NG = B * WIN


def _proj_body(x_ref, w_ref, alf_ref, arf_ref, feat_ref, el_ref, er_ref):
    x = x_ref[...]
    w = w_ref[0]
    feat = jnp.dot(x, w, preferred_element_type=jnp.float32)
    feat_ref[...] = feat
    seg = (lax.broadcasted_iota(jnp.int32, (D_OUT, H), 0) // DH
           == lax.broadcasted_iota(jnp.int32, (D_OUT, H), 1)).astype(jnp.float32)
    el = jnp.dot(feat * alf_ref[0, 0][None, :], seg,
                 preferred_element_type=jnp.float32)
    el_ref[0] = el.T
    er_ref[...] = jnp.dot(feat * arf_ref[0, 0][None, :], seg,
                          preferred_element_type=jnp.float32)


def _project(packed, wstk, alf, arf):
    nblk = NROW // 256
    return pl.pallas_call(
        _proj_body,
        grid=(nblk,),
        in_specs=[
            pl.BlockSpec((256, D_IN), lambda i: (i, 0)),
            pl.BlockSpec((1, D_IN, D_OUT), lambda i: (i // 10, 0, 0)),
            pl.BlockSpec((1, 1, D_OUT), lambda i: (i // 10, 0, 0)),
            pl.BlockSpec((1, 1, D_OUT), lambda i: (i // 10, 0, 0)),
        ],
        out_specs=[
            pl.BlockSpec((256, D_OUT), lambda i: (i, 0)),
            pl.BlockSpec((1, H, 256), lambda i: (i // 10, 0, i % 10)),
            pl.BlockSpec((256, H), lambda i: (i, 0)),
        ],
        out_shape=[
            jax.ShapeDtypeStruct((NROW, D_OUT), jnp.float32),
            jax.ShapeDtypeStruct((2, H, NP), jnp.float32),
            jax.ShapeDtypeStruct((NROW, H), jnp.float32),
        ],
    )(packed, wstk, alf, arf)


def _gat_body(cnt_ref, er_ref, elt_ref, feat_ref, out_ref):
    cnt = cnt_ref[...]
    for h in range(H):
        e = er_ref[:, h][:, None] + elt_ref[0, h, :][None, :]
        e = jnp.maximum(e, 0.2 * e)
        p = cnt * jnp.exp(e)
        esum = jnp.sum(p, axis=1) + 1e-9
        acc = jnp.dot(p, feat_ref[:, h * DH:(h + 1) * DH],
                      preferred_element_type=jnp.float32)
        o = acc / esum[:, None]
        out_ref[:, h * DH:(h + 1) * DH] = jnp.where(o > 0, o, jnp.exp(o) - 1.0)


def _gat_dense(cnt2, er, elt, feat):
    return pl.pallas_call(
        _gat_body,
        grid=(2, NP // 256),
        in_specs=[
            pl.BlockSpec((256, NP), lambda g, d: (g * 10 + d, 0)),
            pl.BlockSpec((256, H), lambda g, d: (g * 10 + d, 0)),
            pl.BlockSpec((1, H, NP), lambda g, d: (g, 0, 0)),
            pl.BlockSpec((NP, D_OUT), lambda g, d: (g, 0)),
        ],
        out_specs=pl.BlockSpec((256, D_OUT), lambda g, d: (g * 10 + d, 0)),
        out_shape=jax.ShapeDtypeStruct((NROW, D_OUT), jnp.float32),
    )(cnt2, er, elt, feat)


def _final_body(gat_ref, bnd_ref, w1_ref, b1_ref, w2_ref, out_ref):
    iota_p = lax.broadcasted_iota(jnp.int32, (NG, NP), 1)
    au = bnd_ref[0, :][:, None]
    bu = bnd_ref[1, :][:, None]
    ai = bnd_ref[2, :][:, None]
    bi = bnd_ref[3, :][:, None]
    mu = ((iota_p >= au) & (iota_p < bu)).astype(jnp.float32)
    mi = ((iota_p >= ai) & (iota_p < bi)).astype(jnp.float32)
    snaps = (jnp.dot(mu, gat_ref[:NP, :], preferred_element_type=jnp.float32)
             + jnp.dot(mi, gat_ref[NP:, :], preferred_element_type=jnp.float32)
             ) * (1.0 / NODE_COUNT)
    q = jnp.tanh(jnp.dot(snaps, w1_ref[...], preferred_element_type=jnp.float32)
                 + b1_ref[...])
    s = jnp.sum(q * w2_ref[...], axis=1)
    er_ = jnp.exp(s)[None, :]
    iota_r = lax.broadcasted_iota(jnp.int32, (B, NG), 1)
    iota_b = lax.broadcasted_iota(jnp.int32, (B, NG), 0)
    bsel = jnp.where(iota_r // WIN == iota_b, er_, 0.0)
    denom = jnp.sum(bsel, axis=1)
    out_ref[...] = jnp.dot(bsel, snaps,
                           preferred_element_type=jnp.float32) / denom[:, None]


def _final(gat, bounds, w1, b1row, w2row):
    return pl.pallas_call(
        _final_body,
        grid=(1,),
        in_specs=[
            pl.BlockSpec((NROW, D_OUT), lambda i: (0, 0)),
            pl.BlockSpec((4, NG), lambda i: (0, 0)),
            pl.BlockSpec((D_OUT, HID), lambda i: (0, 0)),
            pl.BlockSpec((1, HID), lambda i: (0, 0)),
            pl.BlockSpec((1, HID), lambda i: (0, 0)),
        ],
        out_specs=pl.BlockSpec((B, D_OUT), lambda i: (0, 0)),
        out_shape=jax.ShapeDtypeStruct((B, D_OUT), jnp.float32),
    )(gat, bounds, w1, b1row, w2row)


def _cumsum_body(t_ref, cu_ref):
    x = (t_ref[...] == 0).astype(jnp.float32)
    il = lax.broadcasted_iota(jnp.int32, (128, 128), 0)
    jl = lax.broadcasted_iota(jnp.int32, (128, 128), 1)
    lower = (il <= jl).astype(jnp.float32)
    within = jnp.dot(x, lower, preferred_element_type=jnp.float32)
    rowtot = within[:, 127][:, None]
    ir = lax.broadcasted_iota(jnp.int32, (80, 80), 0)
    jr = lax.broadcasted_iota(jnp.int32, (80, 80), 1)
    strict = (ir < jr).astype(jnp.float32)
    carry = jnp.dot(rowtot.reshape(1, 80), strict,
                    preferred_element_type=jnp.float32)
    cu_ref[...] = (within + carry.reshape(80, 1)).astype(jnp.int32)


def _cumsum_tc(types):
    out = pl.pallas_call(
        _cumsum_body,
        grid=(1,),
        in_specs=[pl.BlockSpec((80, 128), lambda i: (0, 0))],
        out_specs=pl.BlockSpec((80, 128), lambda i: (0, 0)),
        out_shape=jax.ShapeDtypeStruct((80, 128), jnp.int32),
    )(types.reshape(80, 128))
    return out.reshape(-1)


_NSL = 8
_QROWS = NP // _NSL
_QW = _QROWS * NP
_SH = _QW + 128
_DUMP = _QW
_EPT = E // 16
_NB = 79
_TSL = _QW // 16
_ZCH = 6400


def _sc_body(table, types, cu_in, src_u, dst_u, src_i, dst_i,
             packed, cnt,
             idx_a, rows_v, dstb, srcb, idx_buf, ones_v, zbuf,
             tvb, cuv, sentb, sidx, svals,
             shared, shared2, sem_g, sem_s, sem_z):
    cid = lax.axis_index("c")
    sid = lax.axis_index("s")

    lanes = lax.broadcasted_iota(jnp.int32, (16,), 0)

    pltpu.sync_copy(types.at[pl.ds(sid * 640, 640)], tvb)
    pltpu.sync_copy(cu_in.at[pl.ds(sid * 640, 640)], cuv)

    for v in range(11):
        sentb[pl.ds(v * 16, 16)] = jnp.full((16,), N, jnp.int32)
    pltpu.sync_copy(sentb, shared2.at[pl.ds(256 + sid * 176, 176)])
    plsc.subcore_barrier()
    is_u = cid == 0
    tsel = jnp.where(is_u, 0, 1)
    for v in range(40):
        t = tvb[pl.ds(v * 16, 16)]
        cug = cuv[pl.ds(v * 16, 16)]
        nvec = (sid * 640 + v * 16) + lanes
        pos = jnp.where(is_u, cug - 1, nvec - cug)
        ok = (t == tsel) & (pos < NP)
        dumpv = (256 + NP + (v % 8) * 16) + lanes
        sidx[v // 8, pl.ds((v % 8) * 16, 16)] = jnp.where(ok, 256 + pos,
                                                          dumpv)
        svals[v // 8, pl.ds((v % 8) * 16, 16)] = nvec
    for j in range(5):
        pltpu.async_copy(svals.at[j], shared2.at[sidx.at[j]], sem_g)
    for j in range(5):
        pltpu.make_async_copy(svals.at[0], shared2.at[sidx.at[0]],
                              sem_g).wait()
    plsc.subcore_barrier()

    for ph in range(2):
        pltpu.sync_copy(shared2.at[pl.ds(256 + sid * 160 + ph * 80, 80)],
                        idx_a)
        pltpu.async_copy(table.at[idx_a], rows_v, sem_g).wait()
        pltpu.sync_copy(
            rows_v, packed.at[pl.ds(cid * NP + sid * 160 + ph * 80, 80)])

    def _fill_ones(j, _):
        ones_v[pl.ds(j * 16, 16)] = jnp.full((16,), 1.0, jnp.float32)
        return _
    lax.fori_loop(0, _NB * 8, _fill_ones, 0)

    def _fill_z(j, _):
        zbuf[pl.ds(j * 16, 16)] = jnp.zeros((16,), jnp.float32)
        return _
    lax.fori_loop(0, _ZCH // 16, _fill_z, 0)

    for t in range((_NB * 128 - _EPT) // 16):
        dstb[pl.ds(_EPT + t * 16, 16)] = jnp.full((16,), -1, jnp.int32)

    for g in range(2):
        dsrc = dst_u if g == 0 else dst_i
        ssrc = src_u if g == 0 else src_i
        pltpu.sync_copy(dsrc.at[pl.ds(sid * _EPT, _EPT)],
                        dstb.at[pl.ds(0, _EPT)])
        pltpu.sync_copy(ssrc.at[pl.ds(sid * _EPT, _EPT)],
                        srcb.at[pl.ds(0, _EPT)])
        for oct_ in range(_NSL // 2):
            q = (_NSL // 2) * cid + oct_
            qlo = q * _QROWS
            nz = _TSL // _ZCH
            for z in range(nz):
                pltpu.async_copy(
                    zbuf, shared.at[pl.ds(sid * _TSL + z * _ZCH, _ZCH)],
                    sem_z)
            for z in range(nz):
                pltpu.make_async_copy(
                    zbuf, shared.at[pl.ds(sid * _TSL, _ZCH)], sem_z).wait()

            def _mk(j, _, qlo=qlo):
                for k in range(8):
                    p0 = j * 128 + k * 16
                    d = dstb[pl.ds(p0, 16)]
                    s = srcb[pl.ds(p0, 16)]
                    ok = (d >= qlo) & (d < qlo + _QROWS)
                    flat = (d - qlo) * NP + s
                    dumpv = (_DUMP + k * 16) + lanes
                    idx_buf[pl.ds(p0, 16)] = jnp.where(ok, flat, dumpv)
                return _
            lax.fori_loop(0, _NB, _mk, 0)
            plsc.subcore_barrier()
            pltpu.async_copy(ones_v, shared.at[idx_buf], sem_s,
                             add=True).wait()
            plsc.subcore_barrier()
            doff = (g * NP + qlo) * NP + sid * _TSL
            pltpu.sync_copy(shared.at[pl.ds(sid * _TSL, _TSL)],
                            cnt.at[pl.ds(doff, _TSL)])
            plsc.subcore_barrier()


def _pack_and_count(table, types, cu, ei_u, ei_i):
    mesh = plsc.VectorSubcoreMesh(core_axis_name="c", subcore_axis_name="s")
    f = pl.kernel(
        _sc_body,
        out_type=[
            jax.ShapeDtypeStruct((NROW, D_IN), jnp.float32),
            jax.ShapeDtypeStruct((2 * NP * NP,), jnp.float32),
        ],
        mesh=mesh,
        scratch_types=[
            pltpu.VMEM((80,), jnp.int32),
            pltpu.VMEM((80, D_IN), jnp.float32),
            pltpu.VMEM((_NB * 128,), jnp.int32),
            pltpu.VMEM((_NB * 128,), jnp.int32),
            pltpu.VMEM((_NB * 128,), jnp.int32),
            pltpu.VMEM((_NB * 128,), jnp.float32),
            pltpu.VMEM((_ZCH,), jnp.float32),
            pltpu.VMEM((640,), jnp.int32),
            pltpu.VMEM((640,), jnp.int32),
            pltpu.VMEM((176,), jnp.int32),
            pltpu.VMEM((5, 128), jnp.int32),
            pltpu.VMEM((5, 128), jnp.int32),
            pltpu.VMEM_SHARED((_SH,), jnp.float32),
            pltpu.VMEM_SHARED((3072,), jnp.int32),
            pltpu.SemaphoreType.DMA,
            pltpu.SemaphoreType.DMA,
            pltpu.SemaphoreType.DMA,
        ],
    )
    packed, cnt = f(table, types, cu, ei_u[0], ei_u[1], ei_i[0], ei_i[1])
    return packed, cnt.reshape(NROW, NP)


def kernel(outputs, type_edges, edge_index_user, edge_index_item,
           hop_embedding, Wg_u, al_u, ar_u, Wg_i, al_i, ar_i,
           sa_u_w1, sa_u_b1, sa_u_w2, sa_i_w1, sa_i_b1, sa_i_w2,
           sn_w1, sn_b1, sn_w2):
    types = type_edges.reshape(-1).astype(jnp.int32)
    table = jnp.concatenate(
        [outputs.reshape(-1, D_IN), jnp.zeros((8, D_IN), jnp.float32)])
    cu = _cumsum_tc(types)

    packed, cnt2 = _pack_and_count(
        table, types, cu, edge_index_user.astype(jnp.int32),
        edge_index_item.astype(jnp.int32))

    cu0 = jnp.concatenate([jnp.zeros((1,), jnp.int32), cu])
    a_u = cu0[:-1:NODE_COUNT]
    b_u = cu0[NODE_COUNT::NODE_COUNT]
    g40 = jnp.arange(NG, dtype=jnp.int32) * NODE_COUNT
    bounds = jnp.stack([a_u, b_u, g40 - a_u, g40 + NODE_COUNT - b_u])

    wstk = jnp.stack([Wg_u, Wg_i])
    alf = jnp.stack([al_u.reshape(1, -1), al_i.reshape(1, -1)])
    arf = jnp.stack([ar_u.reshape(1, -1), ar_i.reshape(1, -1)])

    feat, elt, er = _project(packed, wstk, alf, arf)
    gat = _gat_dense(cnt2, er, elt, feat)
    return _final(gat, bounds, sn_w1, sn_b1.reshape(1, HID),
                  sn_w2.reshape(1, HID))


# speedup vs baseline: 1.7813x; 1.0287x over previous
import functools

import jax
import jax.numpy as jnp
from jax import lax
from jax.experimental import pallas as pl
from jax.experimental.pallas import tpu as pltpu
from jax.experimental.pallas import tpu_sc as plsc

B = 32
WIN = 8
NODE_COUNT = 40
D_IN = 256
H = 8
DH = 64
D_OUT = H * DH
HID = 128
E = 160000
NP = 2560
N = B * WIN * NODE_COUNT
NROW = 2 * NP
NG = B * WIN


def _proj_body(x_ref, w_ref, alf_ref, arf_ref, feat_ref, el_ref, er_ref):
    x = x_ref[...]
    w = w_ref[0]
    feat = jnp.dot(x, w, preferred_element_type=jnp.float32)
    feat_ref[...] = feat
    seg = (lax.broadcasted_iota(jnp.int32, (D_OUT, H), 0) // DH
           == lax.broadcasted_iota(jnp.int32, (D_OUT, H), 1)).astype(jnp.float32)
    el = jnp.dot(feat * alf_ref[0, 0][None, :], seg,
                 preferred_element_type=jnp.float32)
    el_ref[0] = el.T
    er_ref[...] = jnp.dot(feat * arf_ref[0, 0][None, :], seg,
                          preferred_element_type=jnp.float32)


def _project(packed, wstk, alf, arf):
    nblk = NROW // 256
    return pl.pallas_call(
        _proj_body,
        grid=(nblk,),
        in_specs=[
            pl.BlockSpec((256, D_IN), lambda i: (i, 0)),
            pl.BlockSpec((1, D_IN, D_OUT), lambda i: (i // 10, 0, 0)),
            pl.BlockSpec((1, 1, D_OUT), lambda i: (i // 10, 0, 0)),
            pl.BlockSpec((1, 1, D_OUT), lambda i: (i // 10, 0, 0)),
        ],
        out_specs=[
            pl.BlockSpec((256, D_OUT), lambda i: (i, 0)),
            pl.BlockSpec((1, H, 256), lambda i: (i // 10, 0, i % 10)),
            pl.BlockSpec((256, H), lambda i: (i, 0)),
        ],
        out_shape=[
            jax.ShapeDtypeStruct((NROW, D_OUT), jnp.float32),
            jax.ShapeDtypeStruct((2, H, NP), jnp.float32),
            jax.ShapeDtypeStruct((NROW, H), jnp.float32),
        ],
    )(packed, wstk, alf, arf)


def _gat_body(cnt_ref, er_ref, elt_ref, feat_ref, out_ref):
    cnt = cnt_ref[...]
    for h in range(H):
        e = er_ref[:, h][:, None] + elt_ref[0, h, :][None, :]
        e = jnp.maximum(e, 0.2 * e)
        p = cnt * jnp.exp(e)
        esum = jnp.sum(p, axis=1) + 1e-9
        acc = jnp.dot(p, feat_ref[:, h * DH:(h + 1) * DH],
                      preferred_element_type=jnp.float32)
        o = acc / esum[:, None]
        out_ref[:, h * DH:(h + 1) * DH] = jnp.where(o > 0, o, jnp.exp(o) - 1.0)


def _gat_dense(cnt2, er, elt, feat):
    return pl.pallas_call(
        _gat_body,
        grid=(2, NP // 256),
        in_specs=[
            pl.BlockSpec((256, NP), lambda g, d: (g * 10 + d, 0)),
            pl.BlockSpec((256, H), lambda g, d: (g * 10 + d, 0)),
            pl.BlockSpec((1, H, NP), lambda g, d: (g, 0, 0)),
            pl.BlockSpec((NP, D_OUT), lambda g, d: (g, 0)),
        ],
        out_specs=pl.BlockSpec((256, D_OUT), lambda g, d: (g * 10 + d, 0)),
        out_shape=jax.ShapeDtypeStruct((NROW, D_OUT), jnp.float32),
    )(cnt2, er, elt, feat)


def _final_body(gat_ref, bnd_ref, w1_ref, b1_ref, w2_ref, out_ref):
    iota_p = lax.broadcasted_iota(jnp.int32, (NG, NP), 1)
    au = bnd_ref[0, :][:, None]
    bu = bnd_ref[1, :][:, None]
    ai = bnd_ref[2, :][:, None]
    bi = bnd_ref[3, :][:, None]
    mu = ((iota_p >= au) & (iota_p < bu)).astype(jnp.float32)
    mi = ((iota_p >= ai) & (iota_p < bi)).astype(jnp.float32)
    snaps = (jnp.dot(mu, gat_ref[:NP, :], preferred_element_type=jnp.float32)
             + jnp.dot(mi, gat_ref[NP:, :], preferred_element_type=jnp.float32)
             ) * (1.0 / NODE_COUNT)
    q = jnp.tanh(jnp.dot(snaps, w1_ref[...], preferred_element_type=jnp.float32)
                 + b1_ref[...])
    s = jnp.sum(q * w2_ref[...], axis=1)
    er_ = jnp.exp(s)[None, :]
    iota_r = lax.broadcasted_iota(jnp.int32, (B, NG), 1)
    iota_b = lax.broadcasted_iota(jnp.int32, (B, NG), 0)
    bsel = jnp.where(iota_r // WIN == iota_b, er_, 0.0)
    denom = jnp.sum(bsel, axis=1)
    out_ref[...] = jnp.dot(bsel, snaps,
                           preferred_element_type=jnp.float32) / denom[:, None]


def _final(gat, bounds, w1, b1row, w2row):
    return pl.pallas_call(
        _final_body,
        grid=(1,),
        in_specs=[
            pl.BlockSpec((NROW, D_OUT), lambda i: (0, 0)),
            pl.BlockSpec((4, NG), lambda i: (0, 0)),
            pl.BlockSpec((D_OUT, HID), lambda i: (0, 0)),
            pl.BlockSpec((1, HID), lambda i: (0, 0)),
            pl.BlockSpec((1, HID), lambda i: (0, 0)),
        ],
        out_specs=pl.BlockSpec((B, D_OUT), lambda i: (0, 0)),
        out_shape=jax.ShapeDtypeStruct((B, D_OUT), jnp.float32),
    )(gat, bounds, w1, b1row, w2row)


def _cumsum_body(t_ref, cu_ref):
    x = (t_ref[...] == 0).astype(jnp.float32)
    il = lax.broadcasted_iota(jnp.int32, (128, 128), 0)
    jl = lax.broadcasted_iota(jnp.int32, (128, 128), 1)
    lower = (il <= jl).astype(jnp.float32)
    within = jnp.dot(x, lower, preferred_element_type=jnp.float32)
    rowtot = within[:, 127][:, None]
    ir = lax.broadcasted_iota(jnp.int32, (80, 80), 0)
    jr = lax.broadcasted_iota(jnp.int32, (80, 80), 1)
    strict = (ir < jr).astype(jnp.float32)
    carry = jnp.dot(rowtot.reshape(1, 80), strict,
                    preferred_element_type=jnp.float32)
    cu_ref[...] = (within + carry.reshape(80, 1)).astype(jnp.int32)


def _cumsum_tc(types):
    out = pl.pallas_call(
        _cumsum_body,
        grid=(1,),
        in_specs=[pl.BlockSpec((80, 128), lambda i: (0, 0))],
        out_specs=pl.BlockSpec((80, 128), lambda i: (0, 0)),
        out_shape=jax.ShapeDtypeStruct((80, 128), jnp.int32),
    )(types.reshape(80, 128))
    return out.reshape(-1)


_NSL = 8
_QROWS = NP // _NSL
_QW = _QROWS * NP
_SH = _QW + 128
_DUMP = _QW
_EPT = E // 16
_NB = 79
_TSL = _QW // 16
_ZCH = 6400


def _sc_body(table, types, cu_in, src_u, dst_u, src_i, dst_i,
             packed, cnt,
             idx_a, rows_v, dstb, srcb, idx_buf, ones_v, zbuf,
             tvb, cuv, sentb, sidx, svals,
             shared, shared2, sem_g, sem_s, sem_z):
    cid = lax.axis_index("c")
    sid = lax.axis_index("s")

    lanes = lax.broadcasted_iota(jnp.int32, (16,), 0)

    pltpu.sync_copy(types.at[pl.ds(sid * 640, 640)], tvb)
    pltpu.sync_copy(cu_in.at[pl.ds(sid * 640, 640)], cuv)

    for v in range(11):
        sentb[pl.ds(v * 16, 16)] = jnp.full((16,), N, jnp.int32)
    pltpu.sync_copy(sentb, shared2.at[pl.ds(256 + sid * 176, 176)])
    plsc.subcore_barrier()
    is_u = cid == 0
    tsel = jnp.where(is_u, 0, 1)
    for v in range(40):
        t = tvb[pl.ds(v * 16, 16)]
        cug = cuv[pl.ds(v * 16, 16)]
        nvec = (sid * 640 + v * 16) + lanes
        pos = jnp.where(is_u, cug - 1, nvec - cug)
        ok = (t == tsel) & (pos < NP)
        dumpv = (256 + NP + (v % 8) * 16) + lanes
        sidx[v // 8, pl.ds((v % 8) * 16, 16)] = jnp.where(ok, 256 + pos,
                                                          dumpv)
        svals[v // 8, pl.ds((v % 8) * 16, 16)] = nvec
    for j in range(5):
        pltpu.async_copy(svals.at[j], shared2.at[sidx.at[j]], sem_g)
    for j in range(5):
        pltpu.make_async_copy(svals.at[0], shared2.at[sidx.at[0]],
                              sem_g).wait()
    plsc.subcore_barrier()

    for ph in range(2):
        pltpu.sync_copy(shared2.at[pl.ds(256 + sid * 160 + ph * 80, 80)],
                        idx_a)
        pltpu.async_copy(table.at[idx_a], rows_v, sem_g).wait()
        pltpu.sync_copy(
            rows_v, packed.at[pl.ds(cid * NP + sid * 160 + ph * 80, 80)])

    def _fill_ones(j, _):
        ones_v[pl.ds(j * 16, 16)] = jnp.full((16,), 1.0, jnp.float32)
        return _
    lax.fori_loop(0, 8, _fill_ones, 0)

    def _fill_z(j, _):
        zbuf[pl.ds(j * 16, 16)] = jnp.zeros((16,), jnp.float32)
        return _
    lax.fori_loop(0, _ZCH // 16, _fill_z, 0)

    for t in range((_NB * 128 - _EPT) // 16):
        dstb[pl.ds(_EPT + t * 16, 16)] = jnp.full((16,), -1, jnp.int32)

    for g in range(2):
        dsrc = dst_u if g == 0 else dst_i
        ssrc = src_u if g == 0 else src_i
        pltpu.sync_copy(dsrc.at[pl.ds(sid * _EPT, _EPT)],
                        dstb.at[pl.ds(0, _EPT)])
        pltpu.sync_copy(ssrc.at[pl.ds(sid * _EPT, _EPT)],
                        srcb.at[pl.ds(0, _EPT)])
        for oct_ in range(_NSL // 2):
            q = (_NSL // 2) * cid + oct_
            qlo = q * _QROWS
            nz = _TSL // _ZCH
            for z in range(nz):
                pltpu.async_copy(
                    zbuf, shared.at[pl.ds(sid * _TSL + z * _ZCH, _ZCH)],
                    sem_z)
            for z in range(nz):
                pltpu.make_async_copy(
                    zbuf, shared.at[pl.ds(sid * _TSL, _ZCH)], sem_z).wait()

            def _mk(j, _, qlo=qlo):
                for k in range(8):
                    p0 = j * 128 + k * 16
                    d = dstb[pl.ds(p0, 16)]
                    s = srcb[pl.ds(p0, 16)]
                    ok = (d >= qlo) & (d < qlo + _QROWS)
                    flat = (d - qlo) * NP + s
                    dumpv = (_DUMP + k * 16) + lanes
                    idx_buf[j, pl.ds(k * 16, 16)] = jnp.where(ok, flat, dumpv)
                pltpu.async_copy(ones_v, shared.at[idx_buf.at[j]], sem_s,
                                 add=True)
                return _
            lax.fori_loop(0, _NB, _mk, 0)

            def _dr(j, _):
                pltpu.make_async_copy(ones_v, shared.at[idx_buf.at[0]],
                                      sem_s).wait()
                return _
            lax.fori_loop(0, _NB, _dr, 0)
            plsc.subcore_barrier()
            doff = (g * NP + qlo) * NP + sid * _TSL
            pltpu.sync_copy(shared.at[pl.ds(sid * _TSL, _TSL)],
                            cnt.at[pl.ds(doff, _TSL)])
            plsc.subcore_barrier()


def _pack_and_count(table, types, cu, ei_u, ei_i):
    mesh = plsc.VectorSubcoreMesh(core_axis_name="c", subcore_axis_name="s")
    f = pl.kernel(
        _sc_body,
        out_type=[
            jax.ShapeDtypeStruct((NROW, D_IN), jnp.float32),
            jax.ShapeDtypeStruct((2 * NP * NP,), jnp.float32),
        ],
        mesh=mesh,
        scratch_types=[
            pltpu.VMEM((80,), jnp.int32),
            pltpu.VMEM((80, D_IN), jnp.float32),
            pltpu.VMEM((_NB * 128,), jnp.int32),
            pltpu.VMEM((_NB * 128,), jnp.int32),
            pltpu.VMEM((_NB, 128), jnp.int32),
            pltpu.VMEM((128,), jnp.float32),
            pltpu.VMEM((_ZCH,), jnp.float32),
            pltpu.VMEM((640,), jnp.int32),
            pltpu.VMEM((640,), jnp.int32),
            pltpu.VMEM((176,), jnp.int32),
            pltpu.VMEM((5, 128), jnp.int32),
            pltpu.VMEM((5, 128), jnp.int32),
            pltpu.VMEM_SHARED((_SH,), jnp.float32),
            pltpu.VMEM_SHARED((3072,), jnp.int32),
            pltpu.SemaphoreType.DMA,
            pltpu.SemaphoreType.DMA,
            pltpu.SemaphoreType.DMA,
        ],
    )
    packed, cnt = f(table, types, cu, ei_u[0], ei_u[1], ei_i[0], ei_i[1])
    return packed, cnt.reshape(NROW, NP)


def kernel(outputs, type_edges, edge_index_user, edge_index_item,
           hop_embedding, Wg_u, al_u, ar_u, Wg_i, al_i, ar_i,
           sa_u_w1, sa_u_b1, sa_u_w2, sa_i_w1, sa_i_b1, sa_i_w2,
           sn_w1, sn_b1, sn_w2):
    types = type_edges.reshape(-1).astype(jnp.int32)
    table = jnp.concatenate(
        [outputs.reshape(-1, D_IN), jnp.zeros((8, D_IN), jnp.float32)])
    cu = _cumsum_tc(types)

    packed, cnt2 = _pack_and_count(
        table, types, cu, edge_index_user.astype(jnp.int32),
        edge_index_item.astype(jnp.int32))

    cu0 = jnp.concatenate([jnp.zeros((1,), jnp.int32), cu])
    a_u = cu0[:-1:NODE_COUNT]
    b_u = cu0[NODE_COUNT::NODE_COUNT]
    g40 = jnp.arange(NG, dtype=jnp.int32) * NODE_COUNT
    bounds = jnp.stack([a_u, b_u, g40 - a_u, g40 + NODE_COUNT - b_u])

    wstk = jnp.stack([Wg_u, Wg_i])
    alf = jnp.stack([al_u.reshape(1, -1), al_i.reshape(1, -1)])
    arf = jnp.stack([ar_u.reshape(1, -1), ar_i.reshape(1, -1)])

    feat, elt, er = _project(packed, wstk, alf, arf)
    gat = _gat_dense(cnt2, er, elt, feat)
    return _final(gat, bounds, sn_w1, sn_b1.reshape(1, HID),
                  sn_w2.reshape(1, HID))
